# baseline (speedup 1.0000x reference)
import dataclasses
import functools

import jax
import jax.numpy as jnp
from jax import lax
from jax.experimental import pallas as pl
from jax.experimental.pallas import tpu as pltpu
from jax.experimental.pallas import tpu_sc as plsc

EPS = 1e-05
NNZ = 1638400
NFEAT = 1000000
NP = 1 << 20
HALF = NP // 2
NSUB = 16
LANES = 16
C = 12800
PER_SUB = NNZ // NSUB
NCHUNK = PER_SUB // C
DUMROWS = NSUB * LANES
TROWS = HALF + DUMROWS


def kernel(values_x, max_x, bias_x, indices_x):
    idx32 = indices_x.astype(jnp.int32)
    pad = jnp.zeros((NP - NFEAT,), jnp.float32)
    maxp = jnp.concatenate([max_x, pad])
    biasp = jnp.concatenate([bias_x, pad])

    mesh = plsc.VectorSubcoreMesh(core_axis_name="c", subcore_axis_name="s")

    cparams = pltpu.CompilerParams()
    if "needs_layout_passes" in pltpu.CompilerParams.__dataclass_fields__:
        cparams = dataclasses.replace(cparams, needs_layout_passes=False)

    @functools.partial(
        pl.kernel,
        compiler_params=cparams,
        out_type=(
            jax.ShapeDtypeStruct((2, NNZ), jnp.float32),
            jax.ShapeDtypeStruct((2 * NNZ,), jnp.int32),
        ),
        mesh=mesh,
        scratch_types=[
            pltpu.VMEM_SHARED((TROWS,), jnp.float32),
            pltpu.VMEM_SHARED((TROWS,), jnp.float32),
            pltpu.VMEM((C,), jnp.int32),
            pltpu.VMEM((C,), jnp.float32),
            pltpu.VMEM((C,), jnp.int32),
            pltpu.VMEM((C,), jnp.float32),
            pltpu.VMEM((C,), jnp.float32),
            pltpu.VMEM((LANES,), jnp.int32),
            pltpu.SMEM((1,), jnp.int32),
        ],
    )
    def sc_kernel(vals_hbm, idx_hbm, maxp_hbm, biasp_hbm, out_hbm, state_hbm,
                  tmax, tbias, ibuf, vbuf, lbuf, mbuf, bbuf, cvec, cnt):
        cid = lax.axis_index("c")
        sid = lax.axis_index("s")
        lo = cid * HALF

        @pl.when(sid == 0)
        def _():
            cnt[0] = 0

        rows = HALF // NSUB
        g0 = lo + sid * rows
        l0 = sid * rows
        pltpu.sync_copy(maxp_hbm.at[pl.ds(g0, rows)], tmax.at[pl.ds(l0, rows)])
        pltpu.sync_copy(biasp_hbm.at[pl.ds(g0, rows)], tbias.at[pl.ds(l0, rows)])
        plsc.subcore_barrier()

        iota = lax.iota(jnp.int32, LANES)
        dum = HALF + sid * LANES + iota
        base_e = sid * PER_SUB

        def work_pass(first):
            def run(_):
                cvec[...] = jnp.zeros((LANES,), jnp.int32)
                for ch in range(NCHUNK):
                    cb = base_e + ch * C
                    sb = cid * NNZ + cb
                    if first:
                        pltpu.sync_copy(idx_hbm.at[pl.ds(cb, C)], ibuf)
                    else:
                        pltpu.sync_copy(state_hbm.at[pl.ds(sb, C)], lbuf)
                    pltpu.sync_copy(vals_hbm.at[pl.ds(cb, C)], vbuf)

                    if first:
                        @pl.loop(0, C, step=LANES)
                        def _(c0):
                            sl = pl.ds(c0, LANES)
                            li = ibuf.at[sl][...] - lo
                            m = (li >= 0) & (li < HALF)
                            lbuf.at[sl][...] = jnp.where(m, li, dum)

                    pltpu.sync_copy(tmax.at[lbuf], mbuf)

                    @pl.loop(0, C, step=LANES)
                    def _(c0):
                        sl = pl.ds(c0, LANES)
                        v = vbuf.at[sl][...]
                        li = lbuf.at[sl][...]
                        cur = mbuf.at[sl][...]
                        m = li < HALF
                        a = jnp.where(m, jnp.abs(v), -1.0)
                        need = m & (a > cur)
                        mbuf.at[sl][...] = jnp.maximum(a, cur)
                        lbuf.at[sl][...] = jnp.where(need, li, dum)
                        cvec[...] = cvec[...] + jnp.where(need, 1, 0)

                    pltpu.sync_copy(mbuf, tmax.at[lbuf])
                    pltpu.sync_copy(lbuf, state_hbm.at[pl.ds(sb, C)])

                mine = jnp.sum(cvec[...])
                plsc.fetch_and_add(cnt.at[0], mine, subcore_id=0)
                plsc.subcore_barrier()
                total = plsc.fetch_and_add(cnt.at[0], 0, subcore_id=0)
                plsc.subcore_barrier()

                @pl.when(sid == 0)
                def _():
                    cnt[0] = 0

                plsc.subcore_barrier()
                return total

            return run

        total0 = work_pass(True)(0)
        lax.while_loop(lambda t: t > 0, work_pass(False), total0)

        for ch in range(NCHUNK):
            cb = base_e + ch * C
            pltpu.sync_copy(idx_hbm.at[pl.ds(cb, C)], ibuf)
            pltpu.sync_copy(vals_hbm.at[pl.ds(cb, C)], vbuf)

            @pl.loop(0, C, step=LANES)
            def _(c0):
                sl = pl.ds(c0, LANES)
                li = ibuf.at[sl][...] - lo
                m = (li >= 0) & (li < HALF)
                lbuf.at[sl][...] = jnp.where(m, li, dum)

            pltpu.sync_copy(tmax.at[lbuf], mbuf)
            pltpu.sync_copy(tbias.at[lbuf], bbuf)

            @pl.loop(0, C, step=LANES)
            def _(c0):
                sl = pl.ds(c0, LANES)
                v = vbuf.at[sl][...]
                li = lbuf.at[sl][...]
                cur = mbuf.at[sl][...]
                b = bbuf.at[sl][...]
                m = li < HALF
                denom = jnp.maximum(cur, EPS)
                r = jnp.minimum(jnp.maximum(v / denom, -1.0), 1.0) + b
                vbuf.at[sl][...] = jnp.where(m, r, 0.0)

            pltpu.sync_copy(vbuf, out_hbm.at[cid, pl.ds(cb, C)])

    halves, _ = sc_kernel(values_x, idx32, maxp, biasp)

    h3 = halves.reshape(2, NNZ // 128, 128)
    nrows = NNZ // 128
    rblk = 800

    def add_body(x_ref, o_ref):
        o_ref[...] = x_ref[0] + x_ref[1]

    merged = pl.pallas_call(
        add_body,
        out_shape=jax.ShapeDtypeStruct((nrows, 128), jnp.float32),
        grid=(nrows // rblk,),
        in_specs=[pl.BlockSpec((2, rblk, 128), lambda i: (0, i, 0))],
        out_specs=pl.BlockSpec((rblk, 128), lambda i: (i, 0)),
    )(h3)
    return merged.reshape(NNZ)


# speedup vs baseline: 1.0170x; 1.0170x over previous
import dataclasses
import functools

import jax
import jax.numpy as jnp
from jax import lax
from jax.experimental import pallas as pl
from jax.experimental.pallas import tpu as pltpu
from jax.experimental.pallas import tpu_sc as plsc

EPS = 1e-05
NNZ = 1638400
NFEAT = 1000000
NP = 1 << 20
HALF = NP // 2
NSUB = 16
LANES = 16
C = 10240
PER_SUB = NNZ // NSUB
NCHUNK = PER_SUB // C
DUMROWS = NSUB * LANES
TROWS = HALF + DUMROWS


def kernel(values_x, max_x, bias_x, indices_x):
    idx32 = indices_x.astype(jnp.int32)
    pad = jnp.zeros((NP - NFEAT,), jnp.float32)
    maxp = jnp.concatenate([max_x, pad])
    biasp = jnp.concatenate([bias_x, pad])

    mesh = plsc.VectorSubcoreMesh(core_axis_name="c", subcore_axis_name="s")

    cparams = pltpu.CompilerParams()
    if "needs_layout_passes" in pltpu.CompilerParams.__dataclass_fields__:
        cparams = dataclasses.replace(cparams, needs_layout_passes=False)

    @functools.partial(
        pl.kernel,
        compiler_params=cparams,
        out_type=(
            jax.ShapeDtypeStruct((2, NNZ), jnp.float32),
            jax.ShapeDtypeStruct((2 * NNZ,), jnp.int32),
        ),
        mesh=mesh,
        scratch_types=[
            pltpu.VMEM_SHARED((TROWS,), jnp.float32),
            pltpu.VMEM_SHARED((TROWS,), jnp.float32),
            pltpu.VMEM((C,), jnp.int32),
            pltpu.VMEM((C,), jnp.float32),
            pltpu.VMEM((C,), jnp.int32),
            pltpu.VMEM((C,), jnp.float32),
            pltpu.VMEM((C,), jnp.float32),
            pltpu.VMEM((LANES,), jnp.int32),
            pltpu.SMEM((1,), jnp.int32),
        ],
    )
    def sc_kernel(vals_hbm, idx_hbm, maxp_hbm, biasp_hbm, out_hbm, state_hbm,
                  tmax, tbias, ibuf, vbuf, lbuf, mbuf, bbuf, cvec, cnt):
        cid = lax.axis_index("c")
        sid = lax.axis_index("s")
        lo = cid * HALF

        @pl.when(sid == 0)
        def _():
            cnt[0] = 0

        rows = HALF // NSUB
        g0 = lo + sid * rows
        l0 = sid * rows
        pltpu.sync_copy(maxp_hbm.at[pl.ds(g0, rows)], tmax.at[pl.ds(l0, rows)])
        pltpu.sync_copy(biasp_hbm.at[pl.ds(g0, rows)], tbias.at[pl.ds(l0, rows)])
        plsc.subcore_barrier()

        iota = lax.iota(jnp.int32, LANES)
        dum = HALF + sid * LANES + iota
        base_e = sid * PER_SUB

        def work_pass(first):
            def run(_):
                cvec[...] = jnp.zeros((LANES,), jnp.int32)
                for ch in range(NCHUNK):
                    cb = base_e + ch * C
                    sb = cid * NNZ + cb
                    if first:
                        pltpu.sync_copy(idx_hbm.at[pl.ds(cb, C)], ibuf)
                    else:
                        pltpu.sync_copy(state_hbm.at[pl.ds(sb, C)], lbuf)
                    pltpu.sync_copy(vals_hbm.at[pl.ds(cb, C)], vbuf)

                    if first:
                        @pl.loop(0, C, step=LANES)
                        def _(c0):
                            sl = pl.ds(c0, LANES)
                            li = ibuf.at[sl][...] - lo
                            m = (li >= 0) & (li < HALF)
                            lbuf.at[sl][...] = jnp.where(m, li, dum)

                    pltpu.sync_copy(tmax.at[lbuf], mbuf)

                    @pl.loop(0, C, step=LANES)
                    def _(c0):
                        sl = pl.ds(c0, LANES)
                        v = vbuf.at[sl][...]
                        li = lbuf.at[sl][...]
                        cur = mbuf.at[sl][...]
                        m = li < HALF
                        a = jnp.where(m, jnp.abs(v), -1.0)
                        need = m & (a > cur)
                        mbuf.at[sl][...] = jnp.maximum(a, cur)
                        lbuf.at[sl][...] = jnp.where(need, li, dum)
                        cvec[...] = cvec[...] + jnp.where(need, 1, 0)

                    pltpu.sync_copy(mbuf, tmax.at[lbuf])
                    pltpu.sync_copy(lbuf, state_hbm.at[pl.ds(sb, C)])

                mine = jnp.sum(cvec[...])
                plsc.fetch_and_add(cnt.at[0], mine, subcore_id=0)
                plsc.subcore_barrier()
                total = plsc.fetch_and_add(cnt.at[0], 0, subcore_id=0)
                plsc.subcore_barrier()

                @pl.when(sid == 0)
                def _():
                    cnt[0] = 0

                plsc.subcore_barrier()
                return total

            return run

        total0 = work_pass(True)(0)
        lax.while_loop(lambda t: t > 0, work_pass(False), total0)

        for ch in range(NCHUNK):
            cb = base_e + ch * C
            pltpu.sync_copy(idx_hbm.at[pl.ds(cb, C)], ibuf)
            pltpu.sync_copy(vals_hbm.at[pl.ds(cb, C)], vbuf)

            @pl.loop(0, C, step=LANES)
            def _(c0):
                sl = pl.ds(c0, LANES)
                li = ibuf.at[sl][...] - lo
                m = (li >= 0) & (li < HALF)
                lbuf.at[sl][...] = jnp.where(m, li, dum)

            pltpu.sync_copy(tmax.at[lbuf], mbuf)
            pltpu.sync_copy(tbias.at[lbuf], bbuf)

            @pl.loop(0, C, step=LANES)
            def _(c0):
                sl = pl.ds(c0, LANES)
                v = vbuf.at[sl][...]
                li = lbuf.at[sl][...]
                cur = mbuf.at[sl][...]
                b = bbuf.at[sl][...]
                m = li < HALF
                denom = jnp.maximum(cur, EPS)
                r = jnp.minimum(jnp.maximum(v / denom, -1.0), 1.0) + b
                vbuf.at[sl][...] = jnp.where(m, r, 0.0)

            pltpu.sync_copy(vbuf, out_hbm.at[cid, pl.ds(cb, C)])

    halves, _ = sc_kernel(values_x, idx32, maxp, biasp)

    h3 = halves.reshape(2, NNZ // 128, 128)
    nrows = NNZ // 128
    rblk = 800

    def add_body(x_ref, o_ref):
        o_ref[...] = x_ref[0] + x_ref[1]

    merged = pl.pallas_call(
        add_body,
        out_shape=jax.ShapeDtypeStruct((nrows, 128), jnp.float32),
        grid=(nrows // rblk,),
        in_specs=[pl.BlockSpec((2, rblk, 128), lambda i: (0, i, 0))],
        out_specs=pl.BlockSpec((rblk, 128), lambda i: (i, 0)),
    )(h3)
    return merged.reshape(NNZ)
